# baseline (speedup 1.0000x reference)
import jax
import jax.numpy as jnp
from jax import lax
from jax.experimental import pallas as pl
from jax.experimental.pallas import tpu as pltpu

_F32 = jnp.float32
_I32 = jnp.int32

_D = 2048
_LM = 128
_MAX_LM = 64
_B = 4
_K = 64
_NTOK = _B * 4096
_ROWS = _B * _K
_RPS = 16
_NSTEP = _ROWS // _RPS
_SPB = _K // _RPS


def _tc_body(sgr_ref, misc_ref,
             ttt_ref, tp_ref, ema_ref, w_ref, aein_ref, aiin_ref,
             *rest):
    xrefs = rest[:_RPS]
    aeout_ref, aiout_ref, lmout_ref, scalout_ref = rest[_RPS:_RPS + 4]
    acc4_ref, den4_ref = rest[_RPS + 4:]
    s = pl.program_id(0)
    lane = lax.broadcasted_iota(_I32, (1, _LM), 1)

    @pl.when(s == 0)
    def _():
        acc4_ref[...] = jnp.zeros((_B, _D), _F32)
        den4_ref[...] = jnp.zeros((_B, _LM), _F32)

    base = s * _RPS
    b = s // _SPB
    rowacc = jnp.zeros((1, _D), _F32)
    etot = _F32(0.0)
    for k in range(_RPS):
        fj = sgr_ref[base + k]
        trow = ttt_ref[pl.ds(fj // _LM, 1), :]
        impj = jnp.sum(jnp.where(lane == fj % _LM, trow, _F32(0.0)))
        ej = jnp.exp(_F32(5.0) * impj)
        rowacc = rowacc + ej * xrefs[k][0]
        etot = etot + ej
    acc4_ref[pl.ds(b, 1), :] = acc4_ref[pl.ds(b, 1), :] + rowacc
    den4_ref[pl.ds(b, 1), :] = den4_ref[pl.ds(b, 1), :] + etot

    @pl.when(s == _NSTEP - 1)
    def _():
        raw = jnp.zeros((1, _D), _F32)
        for bb in range(_B):
            dbb = jnp.sum(jnp.where(lane == 0, den4_ref[bb:bb + 1, :], _F32(0.0)))
            raw = raw + acc4_ref[bb:bb + 1, :] * (_F32(0.25) / dbb)
        lm = jax.lax.dot_general(
            raw, w_ref[...], (((1,), (1,)), ((), ())),
            preferred_element_type=_F32)
        lmout_ref[...] = lm

        mean_err = jnp.sum(ttt_ref[...]) * _F32(1.0 / _NTOK)
        full_prob = jnp.sum(tp_ref[...]) * _F32(0.25)
        emav = ema_ref[...]
        m = jnp.sum(jnp.where(lane == 0, emav, _F32(0.0)))
        v = jnp.sum(jnp.where(lane == 1, emav, _F32(0.0)))
        delta = mean_err - m
        new_mean = m + _F32(0.05) * delta
        new_var = v * _F32(0.95) + _F32(0.05) * delta * delta
        thr = jnp.maximum(
            new_mean + _F32(0.5) * jnp.sqrt(jnp.maximum(new_var, _F32(1e-8))),
            _F32(0.3))
        skip = jnp.logical_and(mean_err < thr, full_prob < _F32(0.5))
        af = jnp.where(skip, _F32(0.0), _F32(1.0))
        score = mean_err * full_prob + _F32(1e-6)

        n = misc_ref[0]
        aeout_ref[...] = aein_ref[...]
        old = aein_ref[pl.ds(n, 1), :]
        aeout_ref[pl.ds(n, 1), :] = af * lm + (_F32(1.0) - af) * old

        lane64 = lax.broadcasted_iota(_I32, (1, _MAX_LM), 1)
        ai = aiin_ref[...]
        newi = af * score + (_F32(1.0) - af) * ai
        aiout_ref[...] = jnp.where(lane64 == n, newi, ai)

        sv = jnp.where(lane == 0, new_mean,
                       jnp.where(lane == 1, new_var,
                                 jnp.where(lane == 2, af, _F32(0.0))))
        scalout_ref[...] = sv


def kernel(scan_out, ttt_importance, tier_probs, sgr_indices, W_compress,
           archived_embeddings, archived_importance, err_ema_mean, err_ema_var,
           n_archived):
    x3 = scan_out.reshape(_NTOK, 1, _D)
    ttt2 = ttt_importance.reshape(_NTOK // _LM, _LM)
    sgrv = sgr_indices.astype(_I32)
    base = (jnp.arange(_B, dtype=_I32) * 4096)[:, None]
    sgr = (sgrv + base).reshape(_ROWS)
    tp = jnp.zeros((1, _LM), _F32).at[0, :4].set(tier_probs[:, 2].astype(_F32))
    ema = jnp.zeros((1, _LM), _F32).at[0, 0].set(err_ema_mean).at[0, 1].set(err_ema_var)
    misc = jnp.zeros((8,), _I32).at[0].set(jnp.asarray(n_archived, _I32))
    aiin = archived_importance.reshape(1, _MAX_LM)

    const = lambda i, p1, p2: (0, 0)
    in_specs = [
        pl.BlockSpec((_NTOK // _LM, _LM), const),
        pl.BlockSpec((1, _LM), const),
        pl.BlockSpec((1, _LM), const),
        pl.BlockSpec((_LM, _D), const),
        pl.BlockSpec((_MAX_LM, _LM), const),
        pl.BlockSpec((1, _MAX_LM), const),
    ] + [
        pl.BlockSpec((1, 1, _D),
                     (lambda i, p1, p2, k=k: (p1[_RPS * i + k], 0, 0)))
        for k in range(_RPS)
    ]
    out_specs = [
        pl.BlockSpec((_MAX_LM, _LM), const),
        pl.BlockSpec((1, _MAX_LM), const),
        pl.BlockSpec((1, _LM), const),
        pl.BlockSpec((1, _LM), const),
    ]
    grid_spec = pltpu.PrefetchScalarGridSpec(
        num_scalar_prefetch=2,
        grid=(_NSTEP,),
        in_specs=in_specs,
        out_specs=out_specs,
        scratch_shapes=[
            pltpu.VMEM((_B, _D), _F32),
            pltpu.VMEM((_B, _LM), _F32),
        ],
    )
    aeout, aiout, lmout, scal = pl.pallas_call(
        _tc_body,
        grid_spec=grid_spec,
        out_shape=[
            jax.ShapeDtypeStruct((_MAX_LM, _LM), _F32),
            jax.ShapeDtypeStruct((1, _MAX_LM), _F32),
            jax.ShapeDtypeStruct((1, _LM), _F32),
            jax.ShapeDtypeStruct((1, _LM), _F32),
        ],
    )(sgr, misc, ttt2, tp, ema, W_compress,
      archived_embeddings, aiin, *([x3] * _RPS))

    return (aeout, aiout.reshape(_MAX_LM), lmout.reshape(_LM),
            scal[0, 2] > 0.5, scal[0, 0], scal[0, 1])


# speedup vs baseline: 3.5574x; 3.5574x over previous
import jax
import jax.numpy as jnp
from jax import lax
from jax.experimental import pallas as pl
from jax.experimental.pallas import tpu as pltpu

_F32 = jnp.float32
_I32 = jnp.int32

_D = 2048
_LM = 128
_MAX_LM = 64
_B = 4
_K = 64
_NTOK = _B * 4096
_ROWS = _B * _K
_RPS = 16
_NSTEP = _ROWS // _RPS
_SPB = _K // _RPS


def _tc_body(sgr_ref, misc_ref,
             ttt_ref, tp_ref, ema_ref, w_ref, aein_ref, aiin_ref,
             *rest):
    xrefs = rest[:_RPS]
    aeout_ref, aiout_ref, lmout_ref, scalout_ref = rest[_RPS:_RPS + 4]
    acc4_ref, den4_ref = rest[_RPS + 4:]
    s = pl.program_id(0)
    lane = lax.broadcasted_iota(_I32, (1, _LM), 1)

    @pl.when(s == 0)
    def _():
        acc4_ref[...] = jnp.zeros((_B * 8, _D // 8), _F32)
        den4_ref[...] = jnp.zeros((_B, _LM), _F32)

    base = s * _RPS
    b = s // _SPB
    rowacc = jnp.zeros((8, _D // 8), _F32)
    etot = _F32(0.0)
    for k in range(_RPS):
        fj = sgr_ref[base + k]
        trow = ttt_ref[pl.ds(fj // _LM, 1), :]
        impj = jnp.sum(jnp.where(lane == fj % _LM, trow, _F32(0.0)))
        ej = jnp.exp(_F32(5.0) * impj)
        rowacc = rowacc + ej * xrefs[k][0]
        etot = etot + ej
    acc4_ref[pl.ds(b * 8, 8), :] = acc4_ref[pl.ds(b * 8, 8), :] + rowacc
    den4_ref[pl.ds(b, 1), :] = den4_ref[pl.ds(b, 1), :] + etot

    @pl.when(s == _NSTEP - 1)
    def _():
        raw = jnp.zeros((8, _D // 8), _F32)
        for bb in range(_B):
            dbb = jnp.sum(jnp.where(lane == 0, den4_ref[bb:bb + 1, :], _F32(0.0)))
            raw = raw + acc4_ref[8 * bb:8 * bb + 8, :] * (_F32(0.25) / dbb)
        lm = jnp.zeros((1, _LM), _F32)
        for ss in range(8):
            lm = lm + jax.lax.dot_general(
                raw[ss:ss + 1, :], w_ref[:, ss, :], (((1,), (1,)), ((), ())),
                preferred_element_type=_F32)
        lmout_ref[...] = lm

        mean_err = jnp.sum(ttt_ref[...]) * _F32(1.0 / _NTOK)
        full_prob = jnp.sum(tp_ref[...]) * _F32(0.25)
        emav = ema_ref[...]
        m = jnp.sum(jnp.where(lane == 0, emav, _F32(0.0)))
        v = jnp.sum(jnp.where(lane == 1, emav, _F32(0.0)))
        delta = mean_err - m
        new_mean = m + _F32(0.05) * delta
        new_var = v * _F32(0.95) + _F32(0.05) * delta * delta
        thr = jnp.maximum(
            new_mean + _F32(0.5) * jnp.sqrt(jnp.maximum(new_var, _F32(1e-8))),
            _F32(0.3))
        skip = jnp.logical_and(mean_err < thr, full_prob < _F32(0.5))
        af = jnp.where(skip, _F32(0.0), _F32(1.0))
        score = mean_err * full_prob + _F32(1e-6)

        n = misc_ref[0]
        aeout_ref[...] = aein_ref[...]
        old = aein_ref[pl.ds(n, 1), :]
        aeout_ref[pl.ds(n, 1), :] = af * lm + (_F32(1.0) - af) * old

        lane64 = lax.broadcasted_iota(_I32, (1, _MAX_LM), 1)
        ai = aiin_ref[...]
        newi = af * score + (_F32(1.0) - af) * ai
        aiout_ref[...] = jnp.where(lane64 == n, newi, ai)

        sv = jnp.where(lane == 0, new_mean,
                       jnp.where(lane == 1, new_var,
                                 jnp.where(lane == 2, af, _F32(0.0))))
        scalout_ref[...] = sv


def kernel(scan_out, ttt_importance, tier_probs, sgr_indices, W_compress,
           archived_embeddings, archived_importance, err_ema_mean, err_ema_var,
           n_archived):
    x3 = scan_out.reshape(_NTOK, 8, _D // 8)
    w3 = W_compress.reshape(_LM, 8, _D // 8)
    ttt2 = ttt_importance.reshape(_NTOK // _LM, _LM)
    sgrv = sgr_indices.astype(_I32)
    base = (jnp.arange(_B, dtype=_I32) * 4096)[:, None]
    sgr = (sgrv + base).reshape(_ROWS)
    tp = jnp.zeros((1, _LM), _F32).at[0, :4].set(tier_probs[:, 2].astype(_F32))
    ema = jnp.zeros((1, _LM), _F32).at[0, 0].set(err_ema_mean).at[0, 1].set(err_ema_var)
    misc = jnp.zeros((8,), _I32).at[0].set(jnp.asarray(n_archived, _I32))
    aiin = archived_importance.reshape(1, _MAX_LM)

    const = lambda i, p1, p2: (0, 0)
    in_specs = [
        pl.BlockSpec((_NTOK // _LM, _LM), const),
        pl.BlockSpec((1, _LM), const),
        pl.BlockSpec((1, _LM), const),
        pl.BlockSpec((_LM, 8, _D // 8), lambda i, p1, p2: (0, 0, 0)),
        pl.BlockSpec((_MAX_LM, _LM), const),
        pl.BlockSpec((1, _MAX_LM), const),
    ] + [
        pl.BlockSpec((1, 8, _D // 8),
                     (lambda i, p1, p2, k=k: (p1[_RPS * i + k], 0, 0)))
        for k in range(_RPS)
    ]
    out_specs = [
        pl.BlockSpec((_MAX_LM, _LM), const),
        pl.BlockSpec((1, _MAX_LM), const),
        pl.BlockSpec((1, _LM), const),
        pl.BlockSpec((1, _LM), const),
    ]
    grid_spec = pltpu.PrefetchScalarGridSpec(
        num_scalar_prefetch=2,
        grid=(_NSTEP,),
        in_specs=in_specs,
        out_specs=out_specs,
        scratch_shapes=[
            pltpu.VMEM((_B * 8, _D // 8), _F32),
            pltpu.VMEM((_B, _LM), _F32),
        ],
    )
    aeout, aiout, lmout, scal = pl.pallas_call(
        _tc_body,
        grid_spec=grid_spec,
        out_shape=[
            jax.ShapeDtypeStruct((_MAX_LM, _LM), _F32),
            jax.ShapeDtypeStruct((1, _MAX_LM), _F32),
            jax.ShapeDtypeStruct((1, _LM), _F32),
            jax.ShapeDtypeStruct((1, _LM), _F32),
        ],
    )(sgr, misc, ttt2, tp, ema, w3,
      archived_embeddings, aiin, *([x3] * _RPS))

    return (aeout, aiout.reshape(_MAX_LM), lmout.reshape(_LM),
            scal[0, 2] > 0.5, scal[0, 0], scal[0, 1])


# speedup vs baseline: 11.3971x; 3.2038x over previous
import jax
import jax.numpy as jnp
from jax import lax
from jax.experimental import pallas as pl
from jax.experimental.pallas import tpu as pltpu

_F32 = jnp.float32
_I32 = jnp.int32

_D = 2048
_LM = 128
_MAX_LM = 64
_B = 4
_K = 64
_NTOK = _B * 4096
_ROWS = _B * _K
_RPS = 16
_NSTEP = _ROWS // _RPS
_SPB = _K // _RPS


def _tc_body(sgr_ref, misc_ref,
             ttt_ref, tp_ref, ema_ref, w_ref, aein_ref, aiin_ref,
             *rest):
    xrefs = rest[:_RPS]
    aeout_ref, aiout_ref, lmout_ref, scalout_ref = rest[_RPS:_RPS + 4]
    acc4_ref, den4_ref = rest[_RPS + 4:]
    s = pl.program_id(0)
    lane = lax.broadcasted_iota(_I32, (1, _LM), 1)

    @pl.when(s == 0)
    def _():
        acc4_ref[...] = jnp.zeros((_B, _D), _F32)
        den4_ref[...] = jnp.zeros((_B, _LM), _F32)

    base = s * _RPS
    b = s // _SPB
    rowacc = jnp.zeros((1, _D), _F32)
    etot = _F32(0.0)
    for k in range(_RPS):
        fj = sgr_ref[base + k]
        trow = ttt_ref[pl.ds(fj // _LM, 1), :]
        impj = jnp.sum(jnp.where(lane == fj % _LM, trow, _F32(0.0)))
        ej = jnp.exp(_F32(5.0) * impj)
        xk = xrefs[k][pl.ds(fj % 8, 1), :]
        rowacc = rowacc + ej * xk
        etot = etot + ej
    acc4_ref[pl.ds(b, 1), :] = acc4_ref[pl.ds(b, 1), :] + rowacc
    den4_ref[pl.ds(b, 1), :] = den4_ref[pl.ds(b, 1), :] + etot

    @pl.when(s == _NSTEP - 1)
    def _():
        raw = jnp.zeros((1, _D), _F32)
        for bb in range(_B):
            dbb = jnp.sum(jnp.where(lane == 0, den4_ref[bb:bb + 1, :], _F32(0.0)))
            raw = raw + acc4_ref[bb:bb + 1, :] * (_F32(0.25) / dbb)
        lm = jax.lax.dot_general(
            raw, w_ref[...], (((1,), (1,)), ((), ())),
            preferred_element_type=_F32)
        lmout_ref[...] = lm

        mean_err = jnp.sum(ttt_ref[...]) * _F32(1.0 / _NTOK)
        full_prob = jnp.sum(tp_ref[...]) * _F32(0.25)
        emav = ema_ref[...]
        m = jnp.sum(jnp.where(lane == 0, emav, _F32(0.0)))
        v = jnp.sum(jnp.where(lane == 1, emav, _F32(0.0)))
        delta = mean_err - m
        new_mean = m + _F32(0.05) * delta
        new_var = v * _F32(0.95) + _F32(0.05) * delta * delta
        thr = jnp.maximum(
            new_mean + _F32(0.5) * jnp.sqrt(jnp.maximum(new_var, _F32(1e-8))),
            _F32(0.3))
        skip = jnp.logical_and(mean_err < thr, full_prob < _F32(0.5))
        af = jnp.where(skip, _F32(0.0), _F32(1.0))
        score = mean_err * full_prob + _F32(1e-6)

        n = misc_ref[0]
        aeout_ref[...] = aein_ref[...]
        old = aein_ref[pl.ds(n, 1), :]
        aeout_ref[pl.ds(n, 1), :] = af * lm + (_F32(1.0) - af) * old

        lane64 = lax.broadcasted_iota(_I32, (1, _MAX_LM), 1)
        ai = aiin_ref[...]
        newi = af * score + (_F32(1.0) - af) * ai
        aiout_ref[...] = jnp.where(lane64 == n, newi, ai)

        sv = jnp.where(lane == 0, new_mean,
                       jnp.where(lane == 1, new_var,
                                 jnp.where(lane == 2, af, _F32(0.0))))
        scalout_ref[...] = sv


def kernel(scan_out, ttt_importance, tier_probs, sgr_indices, W_compress,
           archived_embeddings, archived_importance, err_ema_mean, err_ema_var,
           n_archived):
    x2 = scan_out.reshape(_NTOK, _D)
    ttt2 = ttt_importance.reshape(_NTOK // _LM, _LM)
    sgrv = sgr_indices.astype(_I32)
    base = (jnp.arange(_B, dtype=_I32) * 4096)[:, None]
    sgr = (sgrv + base).reshape(_ROWS)
    tp = jnp.zeros((1, _LM), _F32).at[0, :4].set(tier_probs[:, 2].astype(_F32))
    ema = jnp.zeros((1, _LM), _F32).at[0, 0].set(err_ema_mean).at[0, 1].set(err_ema_var)
    misc = jnp.zeros((8,), _I32).at[0].set(jnp.asarray(n_archived, _I32))
    aiin = archived_importance.reshape(1, _MAX_LM)

    const = lambda i, p1, p2: (0, 0)
    in_specs = [
        pl.BlockSpec((_NTOK // _LM, _LM), const),
        pl.BlockSpec((1, _LM), const),
        pl.BlockSpec((1, _LM), const),
        pl.BlockSpec((_LM, _D), const),
        pl.BlockSpec((_MAX_LM, _LM), const),
        pl.BlockSpec((1, _MAX_LM), const),
    ] + [
        pl.BlockSpec((8, _D),
                     (lambda i, p1, p2, k=k: (p1[_RPS * i + k] // 8, 0)))
        for k in range(_RPS)
    ]
    out_specs = [
        pl.BlockSpec((_MAX_LM, _LM), const),
        pl.BlockSpec((1, _MAX_LM), const),
        pl.BlockSpec((1, _LM), const),
        pl.BlockSpec((1, _LM), const),
    ]
    grid_spec = pltpu.PrefetchScalarGridSpec(
        num_scalar_prefetch=2,
        grid=(_NSTEP,),
        in_specs=in_specs,
        out_specs=out_specs,
        scratch_shapes=[
            pltpu.VMEM((_B, _D), _F32),
            pltpu.VMEM((_B, _LM), _F32),
        ],
    )
    aeout, aiout, lmout, scal = pl.pallas_call(
        _tc_body,
        grid_spec=grid_spec,
        out_shape=[
            jax.ShapeDtypeStruct((_MAX_LM, _LM), _F32),
            jax.ShapeDtypeStruct((1, _MAX_LM), _F32),
            jax.ShapeDtypeStruct((1, _LM), _F32),
            jax.ShapeDtypeStruct((1, _LM), _F32),
        ],
    )(sgr, misc, ttt2, tp, ema, W_compress,
      archived_embeddings, aiin, *([x2] * _RPS))

    return (aeout, aiout.reshape(_MAX_LM), lmout.reshape(_LM),
            scal[0, 2] > 0.5, scal[0, 0], scal[0, 1])
